# baseline (speedup 1.0000x reference)
import functools

import jax
import jax.numpy as jnp
from jax import lax
from jax.experimental import pallas as pl
from jax.experimental.pallas import tpu as pltpu
from jax.experimental.pallas import tpu_sc as plsc

E = 8
H = 768
T = 2048
TM = 256
NTILES = T // TM
G = NTILES + E - 1
NW = 32
CHUNK = T // NW
LANES = 16


def _router_body(gate_t_ref, dest_ref, prob_ref, offs_ref, ranks_ref):
    gate_t = gate_t_ref[...]
    mx = jnp.max(gate_t, axis=0, keepdims=True)
    s = jnp.sum(jnp.exp(gate_t - mx), axis=0, keepdims=True)
    prob_ref[...] = 1.0 / s

    ioe = lax.broadcasted_iota(jnp.int32, (E, T), 0)
    idx = jnp.min(jnp.where(gate_t == mx, ioe, E), axis=0, keepdims=True)
    oh = (ioe == idx).astype(jnp.float32)

    iu0 = lax.broadcasted_iota(jnp.int32, (128, 128), 0)
    iu1 = lax.broadcasted_iota(jnp.int32, (128, 128), 1)
    upper = (iu0 <= iu1).astype(jnp.float32)
    carry = jnp.zeros((E, 1), jnp.float32)
    for i in range(T // 128):
        blk = oh[:, i * 128:(i + 1) * 128]
        c = jnp.dot(blk, upper, preferred_element_type=jnp.float32) + carry
        ranks_ref[:, i * 128:(i + 1) * 128] = c
        carry = c[:, 127:128]
    counts = carry

    il0 = lax.broadcasted_iota(jnp.int32, (E, E), 0)
    il1 = lax.broadcasted_iota(jnp.int32, (E, E), 1)
    strict = (il0 > il1).astype(jnp.float32)
    offs = jnp.dot(strict, counts, preferred_element_type=jnp.float32,
                   precision=lax.Precision.HIGHEST)

    dest_f = jnp.sum(oh * (offs + ranks_ref[...] - 1.0), axis=0, keepdims=True)
    dest_ref[...] = dest_f.astype(jnp.int32)
    offs9 = jnp.concatenate([jnp.zeros((1, 1), jnp.float32), offs + counts],
                            axis=0).astype(jnp.int32)
    offs_ref[...] = jnp.broadcast_to(offs9, (E + 1, 128))


_router = pl.pallas_call(
    _router_body,
    out_shape=[
        jax.ShapeDtypeStruct((1, T), jnp.int32),
        jax.ShapeDtypeStruct((1, T), jnp.float32),
        jax.ShapeDtypeStruct((E + 1, 128), jnp.int32),
    ],
    scratch_shapes=[pltpu.VMEM((E, T), jnp.float32)],
)


def _gmm_body(tid_ref, eid_ref, off_ref, x_ref, w_ref, b_ref, o_ref):
    g = pl.program_id(0)
    m = tid_ref[g]
    e = eid_ref[g]
    rows = m * TM + lax.broadcasted_iota(jnp.int32, (TM, 1), 0)
    mask = (rows >= off_ref[e]) & (rows < off_ref[e + 1])
    y = jnp.dot(x_ref[...].astype(jnp.bfloat16), w_ref[0],
                preferred_element_type=jnp.float32)
    y = jnp.maximum(y + b_ref[0], 0.0)
    t_prev = tid_ref[jnp.maximum(g - 1, 0)]
    first = jnp.logical_or(g == 0, m != t_prev)
    prev = jnp.where(first, 0.0, o_ref[...])
    o_ref[...] = jnp.where(mask, y, prev)


_gmm = pl.pallas_call(
    _gmm_body,
    grid_spec=pltpu.PrefetchScalarGridSpec(
        num_scalar_prefetch=3,
        grid=(G,),
        in_specs=[
            pl.BlockSpec((TM, H), lambda g, tid, eid, off: (tid[g], 0)),
            pl.BlockSpec((1, H, H), lambda g, tid, eid, off: (eid[g], 0, 0)),
            pl.BlockSpec((1, 1, H), lambda g, tid, eid, off: (eid[g], 0, 0)),
        ],
        out_specs=pl.BlockSpec((TM, H), lambda g, tid, eid, off: (tid[g], 0)),
    ),
    out_shape=jax.ShapeDtypeStruct((T, H), jnp.float32),
)


def _make_sc_kernels():
    mesh = plsc.VectorSubcoreMesh(core_axis_name="c", subcore_axis_name="s")

    @functools.partial(
        pl.kernel,
        mesh=mesh,
        out_type=jax.ShapeDtypeStruct((T, H), jnp.float32),
        scratch_types=[
            pltpu.VMEM((CHUNK,), jnp.int32),
            pltpu.VMEM((CHUNK, H), jnp.float32),
            pltpu.SemaphoreType.DMA,
        ],
    )
    def dispatch(x_hbm, dest_hbm, xs_hbm, idx_v, rows_v, sem):
        wid = lax.axis_index("s") * 2 + lax.axis_index("c")
        base = wid * CHUNK
        pltpu.sync_copy(dest_hbm.at[pl.ds(base, CHUNK)], idx_v)
        pltpu.sync_copy(x_hbm.at[pl.ds(base, CHUNK)], rows_v)
        pltpu.async_copy(rows_v, xs_hbm.at[idx_v], sem).wait()

    @functools.partial(
        pl.kernel,
        mesh=mesh,
        out_type=jax.ShapeDtypeStruct((T, H), jnp.float32),
        scratch_types=[
            pltpu.VMEM((CHUNK,), jnp.int32),
            pltpu.VMEM((CHUNK,), jnp.float32),
            pltpu.VMEM((CHUNK, H), jnp.float32),
            pltpu.SemaphoreType.DMA,
        ],
    )
    def combine(y_hbm, dest_hbm, prob_hbm, out_hbm, idx_v, p_v, rows_v, sem):
        wid = lax.axis_index("s") * 2 + lax.axis_index("c")
        base = wid * CHUNK
        pltpu.sync_copy(dest_hbm.at[pl.ds(base, CHUNK)], idx_v)
        pltpu.sync_copy(prob_hbm.at[pl.ds(base, CHUNK)], p_v)
        pltpu.async_copy(y_hbm.at[idx_v], rows_v, sem).wait()

        def scale_group(q, acc):
            pv = p_v[pl.ds(q * LANES, LANES)]
            for j in range(LANES):
                pr = jnp.broadcast_to(pv[j], (LANES,))
                r = q * LANES + j
                for c in range(H // LANES):
                    sl = pl.ds(c * LANES, LANES)
                    rows_v[r, sl] = rows_v[r, sl] * pr
            return acc

        lax.fori_loop(0, CHUNK // LANES, scale_group, 0)
        pltpu.sync_copy(rows_v, out_hbm.at[pl.ds(base, CHUNK)])

    return dispatch, combine


_make_sc_kernels = functools.cache(_make_sc_kernels)


def kernel(input, gate, We, be):
    dest2, prob2, offs2 = _router(gate.T)
    dest = dest2.reshape(T)
    prob = prob2.reshape(T)
    offs = offs2[:, 0]

    first = offs[:E] // TM
    last = (offs[1:] - 1) // TM
    nv = jnp.maximum(last - first + 1, 0)
    cum = jnp.cumsum(nv)
    gidx = jnp.arange(G, dtype=jnp.int32)
    e_g = jnp.minimum(
        jnp.sum((cum[None, :] <= gidx[:, None]).astype(jnp.int32), axis=1),
        E - 1)
    t_g = jnp.clip(first[e_g] + gidx - (cum - nv)[e_g], 0, NTILES - 1)

    dispatch, combine = _make_sc_kernels()
    xs = dispatch(input, dest)
    ys = _gmm(t_g, e_g, offs, xs, We.astype(jnp.bfloat16), be.reshape(E, 1, H))
    return combine(ys, dest, prob)


# speedup vs baseline: 1.1262x; 1.1262x over previous
import functools

import jax
import jax.numpy as jnp
from jax import lax
from jax.experimental import pallas as pl
from jax.experimental.pallas import tpu as pltpu
from jax.experimental.pallas import tpu_sc as plsc

E = 8
H = 768
T = 2048
TM = 256
NTILES = T // TM
G = NTILES + E - 1
NW = 32
CHUNK = T // NW
LANES = 16


def _router_body(gate_t_ref, dest_ref, prob_ref, offs_ref, ranks_ref):
    gate_t = gate_t_ref[...]
    mx = jnp.max(gate_t, axis=0, keepdims=True)
    s = jnp.sum(jnp.exp(gate_t - mx), axis=0, keepdims=True)
    prob_ref[...] = 1.0 / s

    ioe = lax.broadcasted_iota(jnp.int32, (E, T), 0)
    idx = jnp.min(jnp.where(gate_t == mx, ioe, E), axis=0, keepdims=True)
    oh = (ioe == idx).astype(jnp.float32)

    iu0 = lax.broadcasted_iota(jnp.int32, (128, 128), 0)
    iu1 = lax.broadcasted_iota(jnp.int32, (128, 128), 1)
    upper = (iu0 <= iu1).astype(jnp.float32)
    carry = jnp.zeros((E, 1), jnp.float32)
    for i in range(T // 128):
        blk = oh[:, i * 128:(i + 1) * 128]
        c = jnp.dot(blk, upper, preferred_element_type=jnp.float32) + carry
        ranks_ref[:, i * 128:(i + 1) * 128] = c
        carry = c[:, 127:128]
    counts = carry

    il0 = lax.broadcasted_iota(jnp.int32, (E, E), 0)
    il1 = lax.broadcasted_iota(jnp.int32, (E, E), 1)
    strict = (il0 > il1).astype(jnp.float32)
    offs = jnp.dot(strict, counts, preferred_element_type=jnp.float32,
                   precision=lax.Precision.HIGHEST)

    dest_f = jnp.sum(oh * (offs + ranks_ref[...] - 1.0), axis=0, keepdims=True)
    dest_ref[...] = dest_f.astype(jnp.int32)
    offs9 = jnp.concatenate([jnp.zeros((1, 1), jnp.float32), offs + counts],
                            axis=0).astype(jnp.int32)
    offs_ref[...] = jnp.broadcast_to(offs9, (E + 1, 128))


_router = pl.pallas_call(
    _router_body,
    out_shape=[
        jax.ShapeDtypeStruct((1, T), jnp.int32),
        jax.ShapeDtypeStruct((1, T), jnp.float32),
        jax.ShapeDtypeStruct((E + 1, 128), jnp.int32),
    ],
    scratch_shapes=[pltpu.VMEM((E, T), jnp.float32)],
)


def _gmm_body(tid_ref, eid_ref, off_ref, x_ref, w_ref, b_ref, o_ref):
    g = pl.program_id(0)
    m = tid_ref[g]
    e = eid_ref[g]
    rows = m * TM + lax.broadcasted_iota(jnp.int32, (TM, 1), 0)
    mask = (rows >= off_ref[e]) & (rows < off_ref[e + 1])
    y = jnp.dot(x_ref[...], w_ref[0], preferred_element_type=jnp.float32)
    y = jnp.maximum(y + b_ref[0], 0.0)
    t_prev = tid_ref[jnp.maximum(g - 1, 0)]
    first = jnp.logical_or(g == 0, m != t_prev)
    prev = jnp.where(first, 0.0, o_ref[...])
    o_ref[...] = jnp.where(mask, y, prev)


_gmm = pl.pallas_call(
    _gmm_body,
    grid_spec=pltpu.PrefetchScalarGridSpec(
        num_scalar_prefetch=3,
        grid=(G,),
        in_specs=[
            pl.BlockSpec((TM, H), lambda g, tid, eid, off: (tid[g], 0)),
            pl.BlockSpec((1, H, H), lambda g, tid, eid, off: (eid[g], 0, 0)),
            pl.BlockSpec((1, 1, H), lambda g, tid, eid, off: (eid[g], 0, 0)),
        ],
        out_specs=pl.BlockSpec((TM, H), lambda g, tid, eid, off: (tid[g], 0)),
    ),
    out_shape=jax.ShapeDtypeStruct((T, H), jnp.float32),
)


def _make_sc_kernels():
    mesh = plsc.VectorSubcoreMesh(core_axis_name="c", subcore_axis_name="s")

    @functools.partial(
        pl.kernel,
        mesh=mesh,
        out_type=jax.ShapeDtypeStruct((T, H), jnp.float32),
        scratch_types=[
            pltpu.VMEM((CHUNK,), jnp.int32),
            pltpu.VMEM((CHUNK, H), jnp.float32),
            pltpu.SemaphoreType.DMA,
        ],
    )
    def dispatch(x_hbm, dest_hbm, xs_hbm, idx_v, rows_v, sem):
        wid = lax.axis_index("s") * 2 + lax.axis_index("c")
        base = wid * CHUNK
        pltpu.sync_copy(dest_hbm.at[pl.ds(base, CHUNK)], idx_v)
        pltpu.sync_copy(x_hbm.at[pl.ds(base, CHUNK)], rows_v)
        pltpu.async_copy(rows_v, xs_hbm.at[idx_v], sem).wait()

    @functools.partial(
        pl.kernel,
        mesh=mesh,
        out_type=jax.ShapeDtypeStruct((T, H), jnp.float32),
        scratch_types=[
            pltpu.VMEM((CHUNK,), jnp.int32),
            pltpu.VMEM((CHUNK,), jnp.float32),
            pltpu.VMEM((CHUNK, H), jnp.float32),
            pltpu.SemaphoreType.DMA,
        ],
    )
    def combine(y_hbm, dest_hbm, prob_hbm, out_hbm, idx_v, p_v, rows_v, sem):
        wid = lax.axis_index("s") * 2 + lax.axis_index("c")
        base = wid * CHUNK
        pltpu.sync_copy(dest_hbm.at[pl.ds(base, CHUNK)], idx_v)
        pltpu.sync_copy(prob_hbm.at[pl.ds(base, CHUNK)], p_v)
        pltpu.async_copy(y_hbm.at[idx_v], rows_v, sem).wait()

        def scale_group(q, acc):
            pv = p_v[pl.ds(q * LANES, LANES)]
            for j in range(LANES):
                pr = jnp.broadcast_to(pv[j], (LANES,))
                r = q * LANES + j
                for c in range(H // LANES):
                    sl = pl.ds(c * LANES, LANES)
                    rows_v[r, sl] = rows_v[r, sl] * pr
            return acc

        lax.fori_loop(0, CHUNK // LANES, scale_group, 0)
        pltpu.sync_copy(rows_v, out_hbm.at[pl.ds(base, CHUNK)])

    return dispatch, combine


_make_sc_kernels = functools.cache(_make_sc_kernels)


def kernel(input, gate, We, be):
    dest2, prob2, offs2 = _router(gate.T)
    dest = dest2.reshape(T)
    prob = prob2.reshape(T)
    offs = offs2[:, 0]

    first = offs[:E] // TM
    last = (offs[1:] - 1) // TM
    nv = jnp.maximum(last - first + 1, 0)
    cum = jnp.cumsum(nv)
    gidx = jnp.arange(G, dtype=jnp.int32)
    e_g = jnp.minimum(
        jnp.sum((cum[None, :] <= gidx[:, None]).astype(jnp.int32), axis=1),
        E - 1)
    t_g = jnp.clip(first[e_g] + gidx - (cum - nv)[e_g], 0, NTILES - 1)

    dispatch, combine = _make_sc_kernels()
    xs = dispatch(input, dest)
    ys = _gmm(t_g, e_g, offs, xs, We, be.reshape(E, 1, H))
    return combine(ys, dest, prob)


# speedup vs baseline: 1.1426x; 1.0146x over previous
import functools

import jax
import jax.numpy as jnp
from jax import lax
from jax.experimental import pallas as pl
from jax.experimental.pallas import tpu as pltpu
from jax.experimental.pallas import tpu_sc as plsc

E = 8
H = 768
T = 2048
TM = 256
NTILES = T // TM
G = NTILES + E - 1
NW = 32
CHUNK = T // NW
LANES = 16


def _router_body(gate_t_ref, dest_ref, prob_ref, offs_ref, ranks_ref):
    gate_t = gate_t_ref[...]
    mx = jnp.max(gate_t, axis=0, keepdims=True)
    s = jnp.sum(jnp.exp(gate_t - mx), axis=0, keepdims=True)
    prob_ref[...] = 1.0 / s

    ioe = lax.broadcasted_iota(jnp.int32, (E, T), 0)
    idx = jnp.min(jnp.where(gate_t == mx, ioe, E), axis=0, keepdims=True)
    oh = (ioe == idx).astype(jnp.float32)

    iu0 = lax.broadcasted_iota(jnp.int32, (128, 128), 0)
    iu1 = lax.broadcasted_iota(jnp.int32, (128, 128), 1)
    upper = (iu0 <= iu1).astype(jnp.float32)
    carry = jnp.zeros((E, 1), jnp.float32)
    for i in range(T // 128):
        blk = oh[:, i * 128:(i + 1) * 128]
        c = jnp.dot(blk, upper, preferred_element_type=jnp.float32) + carry
        ranks_ref[:, i * 128:(i + 1) * 128] = c
        carry = c[:, 127:128]
    counts = carry

    il0 = lax.broadcasted_iota(jnp.int32, (E, E), 0)
    il1 = lax.broadcasted_iota(jnp.int32, (E, E), 1)
    strict = (il0 > il1).astype(jnp.float32)
    offs = jnp.dot(strict, counts, preferred_element_type=jnp.float32,
                   precision=lax.Precision.HIGHEST)

    dest_f = jnp.sum(oh * (offs + ranks_ref[...] - 1.0), axis=0, keepdims=True)
    dest_ref[...] = dest_f.astype(jnp.int32)
    offs9 = jnp.concatenate([jnp.zeros((1, 1), jnp.float32), offs + counts],
                            axis=0).astype(jnp.int32)
    offs_ref[...] = jnp.broadcast_to(offs9, (E + 1, 128))


_router = pl.pallas_call(
    _router_body,
    out_shape=[
        jax.ShapeDtypeStruct((1, T), jnp.int32),
        jax.ShapeDtypeStruct((1, T), jnp.float32),
        jax.ShapeDtypeStruct((E + 1, 128), jnp.int32),
    ],
    scratch_shapes=[pltpu.VMEM((E, T), jnp.float32)],
)


def _expert_change(eid_ref, g):
    c = jnp.int32(0)
    for k in range(1, G):
        c = c + ((k <= g) & (eid_ref[k] != eid_ref[k - 1])).astype(jnp.int32)
    return c


def _gmm_body(tid_ref, eid_ref, off_ref, x_ref, w_hbm, b_ref, o_ref,
              w_buf, sems):
    g = pl.program_id(0)
    m = tid_ref[g]
    e = eid_ref[g]

    slot = lax.rem(_expert_change(eid_ref, g), 2)
    new_run = jnp.logical_or(g == 0, e != eid_ref[jnp.maximum(g - 1, 0)])

    @pl.when(g == 0)
    def _():
        pltpu.make_async_copy(w_hbm.at[e], w_buf.at[0], sems.at[0]).start()

    @pl.when(new_run)
    def _():
        pltpu.make_async_copy(w_hbm.at[e], w_buf.at[slot], sems.at[slot]).wait()
        nxt = jnp.int32(-1)
        for k in range(1, G):
            gk = jnp.minimum(g + k, G - 1)
            cand = eid_ref[gk]
            take = jnp.logical_and(nxt < 0, cand != e)
            nxt = jnp.where(take, cand, nxt)

        @pl.when(nxt >= 0)
        def _():
            other = 1 - slot
            pltpu.make_async_copy(
                w_hbm.at[jnp.maximum(nxt, 0)], w_buf.at[other],
                sems.at[other]).start()

    rows = m * TM + lax.broadcasted_iota(jnp.int32, (TM, 1), 0)
    mask = (rows >= off_ref[e]) & (rows < off_ref[e + 1])
    y = jnp.dot(x_ref[...], w_buf[slot], preferred_element_type=jnp.float32)
    y = jnp.maximum(y + b_ref[0], 0.0)
    t_prev = tid_ref[jnp.maximum(g - 1, 0)]
    first = jnp.logical_or(g == 0, m != t_prev)
    prev = jnp.where(first, 0.0, o_ref[...])
    o_ref[...] = jnp.where(mask, y, prev)


_gmm = pl.pallas_call(
    _gmm_body,
    grid_spec=pltpu.PrefetchScalarGridSpec(
        num_scalar_prefetch=3,
        grid=(G,),
        in_specs=[
            pl.BlockSpec((TM, H), lambda g, tid, eid, off: (tid[g], 0)),
            pl.BlockSpec(memory_space=pl.ANY),
            pl.BlockSpec((1, 1, H), lambda g, tid, eid, off: (eid[g], 0, 0)),
        ],
        out_specs=pl.BlockSpec((TM, H), lambda g, tid, eid, off: (tid[g], 0)),
        scratch_shapes=[
            pltpu.VMEM((2, H, H), jnp.float32),
            pltpu.SemaphoreType.DMA((2,)),
        ],
    ),
    out_shape=jax.ShapeDtypeStruct((T, H), jnp.float32),
)


def _make_sc_kernels():
    mesh = plsc.VectorSubcoreMesh(core_axis_name="c", subcore_axis_name="s")

    @functools.partial(
        pl.kernel,
        mesh=mesh,
        out_type=jax.ShapeDtypeStruct((T, H), jnp.float32),
        scratch_types=[
            pltpu.VMEM((CHUNK,), jnp.int32),
            pltpu.VMEM((CHUNK, H), jnp.float32),
            pltpu.SemaphoreType.DMA,
        ],
    )
    def dispatch(x_hbm, dest_hbm, xs_hbm, idx_v, rows_v, sem):
        wid = lax.axis_index("s") * 2 + lax.axis_index("c")
        base = wid * CHUNK
        pltpu.sync_copy(dest_hbm.at[pl.ds(base, CHUNK)], idx_v)
        pltpu.sync_copy(x_hbm.at[pl.ds(base, CHUNK)], rows_v)
        pltpu.async_copy(rows_v, xs_hbm.at[idx_v], sem).wait()

    @functools.partial(
        pl.kernel,
        mesh=mesh,
        out_type=jax.ShapeDtypeStruct((T, H), jnp.float32),
        scratch_types=[
            pltpu.VMEM((CHUNK,), jnp.int32),
            pltpu.VMEM((CHUNK,), jnp.float32),
            pltpu.VMEM((CHUNK, H), jnp.float32),
            pltpu.SemaphoreType.DMA,
        ],
    )
    def combine(y_hbm, dest_hbm, prob_hbm, out_hbm, idx_v, p_v, rows_v, sem):
        wid = lax.axis_index("s") * 2 + lax.axis_index("c")
        base = wid * CHUNK
        pltpu.sync_copy(dest_hbm.at[pl.ds(base, CHUNK)], idx_v)
        pltpu.sync_copy(prob_hbm.at[pl.ds(base, CHUNK)], p_v)
        pltpu.async_copy(y_hbm.at[idx_v], rows_v, sem).wait()

        def scale_group(q, acc):
            pv = p_v[pl.ds(q * LANES, LANES)]
            for j in range(LANES):
                pr = jnp.broadcast_to(pv[j], (LANES,))
                r = q * LANES + j
                for c in range(H // LANES):
                    sl = pl.ds(c * LANES, LANES)
                    rows_v[r, sl] = rows_v[r, sl] * pr
            return acc

        lax.fori_loop(0, CHUNK // LANES, scale_group, 0)
        pltpu.sync_copy(rows_v, out_hbm.at[pl.ds(base, CHUNK)])

    return dispatch, combine


_make_sc_kernels = functools.cache(_make_sc_kernels)


def kernel(input, gate, We, be):
    dest2, prob2, offs2 = _router(gate.T)
    dest = dest2.reshape(T)
    prob = prob2.reshape(T)
    offs = offs2[:, 0]

    first = offs[:E] // TM
    last = (offs[1:] - 1) // TM
    nv = jnp.maximum(last - first + 1, 0)
    cum = jnp.cumsum(nv)
    gidx = jnp.arange(G, dtype=jnp.int32)
    e_g = jnp.minimum(
        jnp.sum((cum[None, :] <= gidx[:, None]).astype(jnp.int32), axis=1),
        E - 1)
    t_g = jnp.clip(first[e_g] + gidx - (cum - nv)[e_g], 0, NTILES - 1)

    dispatch, combine = _make_sc_kernels()
    xs = dispatch(input, dest)
    ys = _gmm(t_g, e_g, offs, xs, We, be.reshape(E, 1, H))
    return combine(ys, dest, prob)


# speedup vs baseline: 1.1431x; 1.0004x over previous
import functools

import jax
import jax.numpy as jnp
from jax import lax
from jax.experimental import pallas as pl
from jax.experimental.pallas import tpu as pltpu
from jax.experimental.pallas import tpu_sc as plsc

E = 8
H = 768
T = 2048
TM = 256
NTILES = T // TM
G = NTILES + E - 1
NW = 32
CHUNK = T // NW
LANES = 16


def _router_body(gate_t_ref, dest_ref, prob_ref, offs_ref, ranks_ref):
    gate_t = gate_t_ref[...]
    mx = jnp.max(gate_t, axis=0, keepdims=True)
    s = jnp.sum(jnp.exp(gate_t - mx), axis=0, keepdims=True)
    prob_ref[...] = 1.0 / s

    ioe = lax.broadcasted_iota(jnp.int32, (E, T), 0)
    idx = jnp.min(jnp.where(gate_t == mx, ioe, E), axis=0, keepdims=True)
    oh = (ioe == idx).astype(jnp.float32)

    iu0 = lax.broadcasted_iota(jnp.int32, (128, 128), 0)
    iu1 = lax.broadcasted_iota(jnp.int32, (128, 128), 1)
    upper = (iu0 <= iu1).astype(jnp.float32)
    carry = jnp.zeros((E, 1), jnp.float32)
    for i in range(T // 128):
        blk = oh[:, i * 128:(i + 1) * 128]
        c = jnp.dot(blk, upper, preferred_element_type=jnp.float32) + carry
        ranks_ref[:, i * 128:(i + 1) * 128] = c
        carry = c[:, 127:128]
    counts = carry

    il0 = lax.broadcasted_iota(jnp.int32, (E, E), 0)
    il1 = lax.broadcasted_iota(jnp.int32, (E, E), 1)
    strict = (il0 > il1).astype(jnp.float32)
    offs = jnp.dot(strict, counts, preferred_element_type=jnp.float32,
                   precision=lax.Precision.HIGHEST)

    dest_f = jnp.sum(oh * (offs + ranks_ref[...] - 1.0), axis=0, keepdims=True)
    dest_ref[...] = dest_f.astype(jnp.int32)
    offs9 = jnp.concatenate([jnp.zeros((1, 1), jnp.float32), offs + counts],
                            axis=0).astype(jnp.int32)
    offs_ref[...] = jnp.broadcast_to(offs9, (E + 1, 128))


_router = pl.pallas_call(
    _router_body,
    out_shape=[
        jax.ShapeDtypeStruct((1, T), jnp.int32),
        jax.ShapeDtypeStruct((1, T), jnp.float32),
        jax.ShapeDtypeStruct((E + 1, 128), jnp.int32),
    ],
    scratch_shapes=[pltpu.VMEM((E, T), jnp.float32)],
)


def _expert_change(eid_ref, g):
    c = jnp.int32(0)
    for k in range(1, G):
        c = c + ((k <= g) & (eid_ref[k] != eid_ref[k - 1])).astype(jnp.int32)
    return c


def _gmm_body(vs_ref, eid_ref, off_ref, x_ref, w_hbm, b_ref, o_ref,
              w_buf, sems):
    m = pl.program_id(0)
    v0 = vs_ref[m]
    v1 = vs_ref[m + 1]
    nvis = vs_ref[NTILES]

    @pl.when(m == 0)
    def _():
        pltpu.make_async_copy(
            w_hbm.at[eid_ref[0]], w_buf.at[0], sems.at[0]).start()

    rows = m * TM + lax.broadcasted_iota(jnp.int32, (TM, 1), 0)

    def visit(v, carry):
        e = eid_ref[v]
        slot = lax.rem(_expert_change(eid_ref, v), 2)
        new_run = jnp.logical_or(v == 0, e != eid_ref[jnp.maximum(v - 1, 0)])

        @pl.when(new_run)
        def _():
            pltpu.make_async_copy(
                w_hbm.at[e], w_buf.at[slot], sems.at[slot]).wait()
            nxt = jnp.int32(-1)
            for k in range(1, G):
                vk = jnp.minimum(v + k, G - 1)
                take = ((nxt < 0) & (eid_ref[vk] != e) & (v + k < nvis))
                nxt = jnp.where(take, eid_ref[vk], nxt)

            @pl.when(nxt >= 0)
            def _():
                other = 1 - slot
                pltpu.make_async_copy(
                    w_hbm.at[jnp.maximum(nxt, 0)], w_buf.at[other],
                    sems.at[other]).start()

        mask = (rows >= off_ref[e]) & (rows < off_ref[e + 1])
        y = jnp.dot(x_ref[...], w_buf[slot], preferred_element_type=jnp.float32)
        y = jnp.maximum(y + b_ref[pl.ds(e, 1), :], 0.0)
        o_ref[...] = jnp.where(mask, y, o_ref[...])
        return carry

    lax.fori_loop(v0, v1, visit, 0)


_gmm = pl.pallas_call(
    _gmm_body,
    grid_spec=pltpu.PrefetchScalarGridSpec(
        num_scalar_prefetch=3,
        grid=(NTILES,),
        in_specs=[
            pl.BlockSpec((TM, H), lambda m, vs, eid, off: (m, 0)),
            pl.BlockSpec(memory_space=pl.ANY),
            pl.BlockSpec((E, H), lambda m, vs, eid, off: (0, 0)),
        ],
        out_specs=pl.BlockSpec((TM, H), lambda m, vs, eid, off: (m, 0)),
        scratch_shapes=[
            pltpu.VMEM((2, H, H), jnp.float32),
            pltpu.SemaphoreType.DMA((2,)),
        ],
    ),
    out_shape=jax.ShapeDtypeStruct((T, H), jnp.float32),
)


def _make_sc_kernels():
    mesh = plsc.VectorSubcoreMesh(core_axis_name="c", subcore_axis_name="s")

    @functools.partial(
        pl.kernel,
        mesh=mesh,
        out_type=jax.ShapeDtypeStruct((T, H), jnp.float32),
        scratch_types=[
            pltpu.VMEM((CHUNK,), jnp.int32),
            pltpu.VMEM((CHUNK, H), jnp.float32),
            pltpu.SemaphoreType.DMA,
        ],
    )
    def dispatch(x_hbm, dest_hbm, xs_hbm, idx_v, rows_v, sem):
        wid = lax.axis_index("s") * 2 + lax.axis_index("c")
        base = wid * CHUNK
        pltpu.sync_copy(dest_hbm.at[pl.ds(base, CHUNK)], idx_v)
        pltpu.sync_copy(x_hbm.at[pl.ds(base, CHUNK)], rows_v)
        pltpu.async_copy(rows_v, xs_hbm.at[idx_v], sem).wait()

    @functools.partial(
        pl.kernel,
        mesh=mesh,
        out_type=jax.ShapeDtypeStruct((T, H), jnp.float32),
        scratch_types=[
            pltpu.VMEM((CHUNK,), jnp.int32),
            pltpu.VMEM((CHUNK,), jnp.float32),
            pltpu.VMEM((CHUNK, H), jnp.float32),
            pltpu.SemaphoreType.DMA,
        ],
    )
    def combine(y_hbm, dest_hbm, prob_hbm, out_hbm, idx_v, p_v, rows_v, sem):
        wid = lax.axis_index("s") * 2 + lax.axis_index("c")
        base = wid * CHUNK
        pltpu.sync_copy(dest_hbm.at[pl.ds(base, CHUNK)], idx_v)
        pltpu.sync_copy(prob_hbm.at[pl.ds(base, CHUNK)], p_v)
        pltpu.async_copy(y_hbm.at[idx_v], rows_v, sem).wait()

        def scale_group(q, acc):
            pv = p_v[pl.ds(q * LANES, LANES)]
            for j in range(LANES):
                pr = jnp.broadcast_to(pv[j], (LANES,))
                r = q * LANES + j
                for c in range(H // LANES):
                    sl = pl.ds(c * LANES, LANES)
                    rows_v[r, sl] = rows_v[r, sl] * pr
            return acc

        lax.fori_loop(0, CHUNK // LANES, scale_group, 0)
        pltpu.sync_copy(rows_v, out_hbm.at[pl.ds(base, CHUNK)])

    return dispatch, combine


_make_sc_kernels = functools.cache(_make_sc_kernels)


def kernel(input, gate, We, be):
    dest2, prob2, offs2 = _router(gate.T)
    dest = dest2.reshape(T)
    prob = prob2.reshape(T)
    offs = offs2[:, 0]

    first = offs[:E] // TM
    last = (offs[1:] - 1) // TM
    nv = jnp.maximum(last - first + 1, 0)
    cum = jnp.cumsum(nv)
    gidx = jnp.arange(G, dtype=jnp.int32)
    e_g = jnp.minimum(
        jnp.sum((cum[None, :] <= gidx[:, None]).astype(jnp.int32), axis=1),
        E - 1)
    t_g = jnp.clip(first[e_g] + gidx - (cum - nv)[e_g], 0, NTILES - 1)
    nvis = cum[E - 1]
    marr = jnp.arange(NTILES + 1, dtype=jnp.int32)
    vs = jnp.sum(((gidx[None, :] < nvis) & (t_g[None, :] < marr[:, None]))
                 .astype(jnp.int32), axis=1)

    dispatch, combine = _make_sc_kernels()
    xs = dispatch(input, dest)
    ys = _gmm(vs, e_g, offs, xs, We, be)
    return combine(ys, dest, prob)


# speedup vs baseline: 1.1821x; 1.0341x over previous
import functools

import jax
import jax.numpy as jnp
from jax import lax
from jax.experimental import pallas as pl
from jax.experimental.pallas import tpu as pltpu
from jax.experimental.pallas import tpu_sc as plsc

E = 8
H = 768
T = 2048
TM = 256
NTILES = T // TM
G = NTILES + E - 1
NW = 32
CHUNK = T // NW
LANES = 16


def _router_body(gate_t_ref, dest_ref, prob_ref, offs_ref, ranks_ref):
    gate_t = gate_t_ref[...]
    mx = jnp.max(gate_t, axis=0, keepdims=True)
    s = jnp.sum(jnp.exp(gate_t - mx), axis=0, keepdims=True)
    prob_ref[...] = 1.0 / s

    ioe = lax.broadcasted_iota(jnp.int32, (E, T), 0)
    idx = jnp.min(jnp.where(gate_t == mx, ioe, E), axis=0, keepdims=True)
    oh = (ioe == idx).astype(jnp.float32)

    iu0 = lax.broadcasted_iota(jnp.int32, (128, 128), 0)
    iu1 = lax.broadcasted_iota(jnp.int32, (128, 128), 1)
    upper = (iu0 <= iu1).astype(jnp.float32)
    carry = jnp.zeros((E, 1), jnp.float32)
    for i in range(T // 128):
        blk = oh[:, i * 128:(i + 1) * 128]
        c = jnp.dot(blk, upper, preferred_element_type=jnp.float32) + carry
        ranks_ref[:, i * 128:(i + 1) * 128] = c
        carry = c[:, 127:128]
    counts = carry

    il0 = lax.broadcasted_iota(jnp.int32, (E, E), 0)
    il1 = lax.broadcasted_iota(jnp.int32, (E, E), 1)
    strict = (il0 > il1).astype(jnp.float32)
    offs = jnp.dot(strict, counts, preferred_element_type=jnp.float32,
                   precision=lax.Precision.HIGHEST)

    dest_f = jnp.sum(oh * (offs + ranks_ref[...] - 1.0), axis=0, keepdims=True)
    dest_ref[...] = dest_f.astype(jnp.int32)
    offs9 = jnp.concatenate([jnp.zeros((1, 1), jnp.float32), offs + counts],
                            axis=0).astype(jnp.int32)
    offs_ref[...] = jnp.broadcast_to(offs9, (E + 1, 128))


_router = pl.pallas_call(
    _router_body,
    out_shape=[
        jax.ShapeDtypeStruct((1, T), jnp.int32),
        jax.ShapeDtypeStruct((1, T), jnp.float32),
        jax.ShapeDtypeStruct((E + 1, 128), jnp.int32),
    ],
    scratch_shapes=[pltpu.VMEM((E, T), jnp.float32)],
)


def _gmm_body(vs_ref, eid_ref, off_ref, x_ref, w_hbm, b_ref, o_ref,
              w_buf, sems):
    m = pl.program_id(0)
    v0 = vs_ref[m]
    v1 = vs_ref[m + 1]

    @pl.when(m == 0)
    def _():
        for ee in range(E):
            pltpu.make_async_copy(
                w_hbm.at[ee], w_buf.at[ee], sems.at[ee]).start()

    rows = m * TM + lax.broadcasted_iota(jnp.int32, (TM, 1), 0)

    def visit(v, carry):
        e = eid_ref[v]
        new_run = jnp.logical_or(v == 0, e != eid_ref[jnp.maximum(v - 1, 0)])

        @pl.when(new_run)
        def _():
            pltpu.make_async_copy(
                w_hbm.at[e], w_buf.at[e], sems.at[e]).wait()

        mask = (rows >= off_ref[e]) & (rows < off_ref[e + 1])
        y = jnp.dot(x_ref[...], w_buf[e], preferred_element_type=jnp.float32)
        y = jnp.maximum(y + b_ref[pl.ds(e, 1), :], 0.0)
        o_ref[...] = jnp.where(mask, y, o_ref[...])
        return carry

    lax.fori_loop(v0, v1, visit, 0)

    @pl.when(m == NTILES - 1)
    def _():
        for ee in range(E):
            @pl.when(off_ref[ee] == off_ref[ee + 1])
            def _():
                pltpu.make_async_copy(
                    w_hbm.at[ee], w_buf.at[ee], sems.at[ee]).wait()


_gmm = pl.pallas_call(
    _gmm_body,
    grid_spec=pltpu.PrefetchScalarGridSpec(
        num_scalar_prefetch=3,
        grid=(NTILES,),
        in_specs=[
            pl.BlockSpec((TM, H), lambda m, vs, eid, off: (m, 0)),
            pl.BlockSpec(memory_space=pl.ANY),
            pl.BlockSpec((E, H), lambda m, vs, eid, off: (0, 0)),
        ],
        out_specs=pl.BlockSpec((TM, H), lambda m, vs, eid, off: (m, 0)),
        scratch_shapes=[
            pltpu.VMEM((E, H, H), jnp.float32),
            pltpu.SemaphoreType.DMA((E,)),
        ],
    ),
    out_shape=jax.ShapeDtypeStruct((T, H), jnp.float32),
)


def _make_sc_kernels():
    mesh = plsc.VectorSubcoreMesh(core_axis_name="c", subcore_axis_name="s")

    @functools.partial(
        pl.kernel,
        mesh=mesh,
        out_type=jax.ShapeDtypeStruct((T, H), jnp.float32),
        scratch_types=[
            pltpu.VMEM((CHUNK,), jnp.int32),
            pltpu.VMEM((CHUNK, H), jnp.float32),
            pltpu.SemaphoreType.DMA,
        ],
    )
    def dispatch(x_hbm, dest_hbm, xs_hbm, idx_v, rows_v, sem):
        wid = lax.axis_index("s") * 2 + lax.axis_index("c")
        base = wid * CHUNK
        pltpu.sync_copy(dest_hbm.at[pl.ds(base, CHUNK)], idx_v)
        pltpu.sync_copy(x_hbm.at[pl.ds(base, CHUNK)], rows_v)
        pltpu.async_copy(rows_v, xs_hbm.at[idx_v], sem).wait()

    @functools.partial(
        pl.kernel,
        mesh=mesh,
        out_type=jax.ShapeDtypeStruct((T, H), jnp.float32),
        scratch_types=[
            pltpu.VMEM((CHUNK,), jnp.int32),
            pltpu.VMEM((CHUNK,), jnp.float32),
            pltpu.VMEM((CHUNK, H), jnp.float32),
            pltpu.SemaphoreType.DMA,
        ],
    )
    def combine(y_hbm, dest_hbm, prob_hbm, out_hbm, idx_v, p_v, rows_v, sem):
        wid = lax.axis_index("s") * 2 + lax.axis_index("c")
        base = wid * CHUNK
        pltpu.sync_copy(dest_hbm.at[pl.ds(base, CHUNK)], idx_v)
        pltpu.sync_copy(prob_hbm.at[pl.ds(base, CHUNK)], p_v)
        pltpu.async_copy(y_hbm.at[idx_v], rows_v, sem).wait()

        def scale_group(q, acc):
            pv = p_v[pl.ds(q * LANES, LANES)]
            for j in range(LANES):
                pr = jnp.broadcast_to(pv[j], (LANES,))
                r = q * LANES + j
                for c in range(H // LANES):
                    sl = pl.ds(c * LANES, LANES)
                    rows_v[r, sl] = rows_v[r, sl] * pr
            return acc

        lax.fori_loop(0, CHUNK // LANES, scale_group, 0)
        pltpu.sync_copy(rows_v, out_hbm.at[pl.ds(base, CHUNK)])

    return dispatch, combine


_make_sc_kernels = functools.cache(_make_sc_kernels)


def kernel(input, gate, We, be):
    dest2, prob2, offs2 = _router(gate.T)
    dest = dest2.reshape(T)
    prob = prob2.reshape(T)
    offs = offs2[:, 0]

    first = offs[:E] // TM
    last = (offs[1:] - 1) // TM
    nv = jnp.maximum(last - first + 1, 0)
    cum = jnp.cumsum(nv)
    gidx = jnp.arange(G, dtype=jnp.int32)
    e_g = jnp.minimum(
        jnp.sum((cum[None, :] <= gidx[:, None]).astype(jnp.int32), axis=1),
        E - 1)
    t_g = jnp.clip(first[e_g] + gidx - (cum - nv)[e_g], 0, NTILES - 1)
    nvis = cum[E - 1]
    marr = jnp.arange(NTILES + 1, dtype=jnp.int32)
    vs = jnp.sum(((gidx[None, :] < nvis) & (t_g[None, :] < marr[:, None]))
                 .astype(jnp.int32), axis=1)

    dispatch, combine = _make_sc_kernels()
    xs = dispatch(input, dest)
    ys = _gmm(vs, e_g, offs, xs, We, be)
    return combine(ys, dest, prob)


# speedup vs baseline: 1.1909x; 1.0075x over previous
import functools

import jax
import jax.numpy as jnp
from jax import lax
from jax.experimental import pallas as pl
from jax.experimental.pallas import tpu as pltpu
from jax.experimental.pallas import tpu_sc as plsc

E = 8
H = 768
T = 2048
TM = 256
NTILES = T // TM
G = NTILES + E - 1
NW = 32
CHUNK = T // NW
LANES = 16


def _router_body(gate_t_ref, dest_ref, prob_ref, offs_ref, ranks_ref):
    gate_t = gate_t_ref[...]
    mx = jnp.max(gate_t, axis=0, keepdims=True)
    s = jnp.sum(jnp.exp(gate_t - mx), axis=0, keepdims=True)
    prob_ref[...] = 1.0 / s

    ioe = lax.broadcasted_iota(jnp.int32, (E, T), 0)
    idx = jnp.min(jnp.where(gate_t == mx, ioe, E), axis=0, keepdims=True)
    oh = (ioe == idx).astype(jnp.float32)

    iu0 = lax.broadcasted_iota(jnp.int32, (128, 128), 0)
    iu1 = lax.broadcasted_iota(jnp.int32, (128, 128), 1)
    upper = (iu0 <= iu1).astype(jnp.float32)
    carry = jnp.zeros((E, 1), jnp.float32)
    for i in range(T // 128):
        blk = oh[:, i * 128:(i + 1) * 128]
        c = jnp.dot(blk, upper, preferred_element_type=jnp.float32) + carry
        ranks_ref[:, i * 128:(i + 1) * 128] = c
        carry = c[:, 127:128]
    counts = carry

    il0 = lax.broadcasted_iota(jnp.int32, (E, E), 0)
    il1 = lax.broadcasted_iota(jnp.int32, (E, E), 1)
    strict = (il0 > il1).astype(jnp.float32)
    offs = jnp.dot(strict, counts, preferred_element_type=jnp.float32,
                   precision=lax.Precision.HIGHEST)

    dest_f = jnp.sum(oh * (offs + ranks_ref[...] - 1.0), axis=0, keepdims=True)
    dest_ref[...] = dest_f.astype(jnp.int32)
    offs9 = jnp.concatenate([jnp.zeros((1, 1), jnp.float32), offs + counts],
                            axis=0).astype(jnp.int32)
    offs_ref[...] = jnp.broadcast_to(offs9, (E + 1, 128))


_router = pl.pallas_call(
    _router_body,
    out_shape=[
        jax.ShapeDtypeStruct((1, T), jnp.int32),
        jax.ShapeDtypeStruct((1, T), jnp.float32),
        jax.ShapeDtypeStruct((E + 1, 128), jnp.int32),
    ],
    scratch_shapes=[pltpu.VMEM((E, T), jnp.float32)],
)


def _gmm_body(vs_ref, eid_ref, off_ref, x_ref, w_hbm, b_ref, o_ref,
              w_buf, sems):
    m = pl.program_id(0)
    v0 = vs_ref[m]
    v1 = vs_ref[m + 1]

    @pl.when(m == 0)
    def _():
        for ee in range(E):
            pltpu.make_async_copy(
                w_hbm.at[ee], w_buf.at[ee], sems.at[ee]).start()

    rows = m * TM + lax.broadcasted_iota(jnp.int32, (TM, 1), 0)

    def visit(v, carry):
        e = eid_ref[v]
        new_run = jnp.logical_or(v == 0, e != eid_ref[jnp.maximum(v - 1, 0)])

        @pl.when(new_run)
        def _():
            pltpu.make_async_copy(
                w_hbm.at[e], w_buf.at[e], sems.at[e]).wait()

        mask = (rows >= off_ref[e]) & (rows < off_ref[e + 1])
        y = jnp.dot(x_ref[...], w_buf[e], preferred_element_type=jnp.float32)
        y = jnp.maximum(y + b_ref[pl.ds(e, 1), :], 0.0)
        o_ref[...] = jnp.where(mask, y, o_ref[...])
        return carry

    lax.fori_loop(v0, v1, visit, 0)

    @pl.when(m == NTILES - 1)
    def _():
        for ee in range(E):
            @pl.when(off_ref[ee] == off_ref[ee + 1])
            def _():
                pltpu.make_async_copy(
                    w_hbm.at[ee], w_buf.at[ee], sems.at[ee]).wait()


_gmm = pl.pallas_call(
    _gmm_body,
    grid_spec=pltpu.PrefetchScalarGridSpec(
        num_scalar_prefetch=3,
        grid=(NTILES,),
        in_specs=[
            pl.BlockSpec((TM, H), lambda m, vs, eid, off: (m, 0)),
            pl.BlockSpec(memory_space=pl.ANY),
            pl.BlockSpec((E, H), lambda m, vs, eid, off: (0, 0)),
        ],
        out_specs=pl.BlockSpec((TM, H), lambda m, vs, eid, off: (m, 0)),
        scratch_shapes=[
            pltpu.VMEM((E, H, H), jnp.float32),
            pltpu.SemaphoreType.DMA((E,)),
        ],
    ),
    out_shape=jax.ShapeDtypeStruct((T, H), jnp.float32),
)


def _make_sc_kernels():
    mesh = plsc.VectorSubcoreMesh(core_axis_name="c", subcore_axis_name="s")

    @functools.partial(
        pl.kernel,
        mesh=mesh,
        out_type=jax.ShapeDtypeStruct((T, H), jnp.float32),
        scratch_types=[
            pltpu.VMEM((CHUNK,), jnp.int32),
            pltpu.VMEM((CHUNK, H), jnp.float32),
            pltpu.SemaphoreType.DMA,
        ],
    )
    def dispatch(x_hbm, dest_hbm, xs_hbm, idx_v, rows_v, sem):
        wid = lax.axis_index("s") * 2 + lax.axis_index("c")
        base = wid * CHUNK
        pltpu.sync_copy(dest_hbm.at[pl.ds(base, CHUNK)], idx_v)
        pltpu.sync_copy(x_hbm.at[pl.ds(base, CHUNK)], rows_v)
        pltpu.async_copy(rows_v, xs_hbm.at[idx_v], sem).wait()

    HALF = CHUNK // 2

    @functools.partial(
        pl.kernel,
        mesh=mesh,
        out_type=jax.ShapeDtypeStruct((T, H), jnp.float32),
        scratch_types=[
            pltpu.VMEM((HALF,), jnp.int32),
            pltpu.VMEM((HALF,), jnp.int32),
            pltpu.VMEM((CHUNK,), jnp.float32),
            pltpu.VMEM((HALF, H), jnp.float32),
            pltpu.VMEM((HALF, H), jnp.float32),
            pltpu.SemaphoreType.DMA,
            pltpu.SemaphoreType.DMA,
            pltpu.SemaphoreType.DMA,
        ],
    )
    def combine(y_hbm, dest_hbm, prob_hbm, out_hbm,
                idx0, idx1, p_v, rows0, rows1, sem0, sem1, wsem):
        wid = lax.axis_index("s") * 2 + lax.axis_index("c")
        base = wid * CHUNK
        pltpu.sync_copy(dest_hbm.at[pl.ds(base, HALF)], idx0)
        g0 = pltpu.make_async_copy(y_hbm.at[idx0], rows0, sem0)
        g0.start()
        pltpu.sync_copy(dest_hbm.at[pl.ds(base + HALF, HALF)], idx1)
        g1 = pltpu.make_async_copy(y_hbm.at[idx1], rows1, sem1)
        g1.start()
        pltpu.sync_copy(prob_hbm.at[pl.ds(base, CHUNK)], p_v)

        def scale_group(rows_v, qoff):
            def body(q, acc):
                pv = p_v[pl.ds(qoff + q * LANES, LANES)]
                for j in range(LANES):
                    pr = jnp.broadcast_to(pv[j], (LANES,))
                    r = q * LANES + j
                    for c in range(H // LANES):
                        sl = pl.ds(c * LANES, LANES)
                        rows_v[r, sl] = rows_v[r, sl] * pr
                return acc
            return body

        g0.wait()
        lax.fori_loop(0, HALF // LANES, scale_group(rows0, 0), 0)
        w0 = pltpu.make_async_copy(rows0, out_hbm.at[pl.ds(base, HALF)], wsem)
        w0.start()
        g1.wait()
        lax.fori_loop(0, HALF // LANES, scale_group(rows1, HALF), 0)
        pltpu.sync_copy(rows1, out_hbm.at[pl.ds(base + HALF, HALF)])
        w0.wait()

    return dispatch, combine


_make_sc_kernels = functools.cache(_make_sc_kernels)


def kernel(input, gate, We, be):
    dest2, prob2, offs2 = _router(gate.T)
    dest = dest2.reshape(T)
    prob = prob2.reshape(T)
    offs = offs2[:, 0]

    first = offs[:E] // TM
    last = (offs[1:] - 1) // TM
    nv = jnp.maximum(last - first + 1, 0)
    cum = jnp.cumsum(nv)
    gidx = jnp.arange(G, dtype=jnp.int32)
    e_g = jnp.minimum(
        jnp.sum((cum[None, :] <= gidx[:, None]).astype(jnp.int32), axis=1),
        E - 1)
    t_g = jnp.clip(first[e_g] + gidx - (cum - nv)[e_g], 0, NTILES - 1)
    nvis = cum[E - 1]
    marr = jnp.arange(NTILES + 1, dtype=jnp.int32)
    vs = jnp.sum(((gidx[None, :] < nvis) & (t_g[None, :] < marr[:, None]))
                 .astype(jnp.int32), axis=1)

    dispatch, combine = _make_sc_kernels()
    xs = dispatch(input, dest)
    ys = _gmm(vs, e_g, offs, xs, We, be)
    return combine(ys, dest, prob)


# speedup vs baseline: 1.2103x; 1.0163x over previous
import functools

import jax
import jax.numpy as jnp
from jax import lax
from jax.experimental import pallas as pl
from jax.experimental.pallas import tpu as pltpu
from jax.experimental.pallas import tpu_sc as plsc

E = 8
H = 768
T = 2048
TM = 256
NTILES = T // TM
G = NTILES + E - 1
NW = 32
CHUNK = T // NW
LANES = 16


def _router_body(gate_t_ref, dest_ref, prob_ref, offs_ref, ranks_ref):
    gate_t = gate_t_ref[...]
    mx = jnp.max(gate_t, axis=0, keepdims=True)
    s = jnp.sum(jnp.exp(gate_t - mx), axis=0, keepdims=True)
    prob_ref[...] = 1.0 / s

    ioe = lax.broadcasted_iota(jnp.int32, (E, T), 0)
    idx = jnp.min(jnp.where(gate_t == mx, ioe, E), axis=0, keepdims=True)
    oh = (ioe == idx).astype(jnp.float32)

    iu0 = lax.broadcasted_iota(jnp.int32, (128, 128), 0)
    iu1 = lax.broadcasted_iota(jnp.int32, (128, 128), 1)
    upper = (iu0 <= iu1).astype(jnp.float32)
    carry = jnp.zeros((E, 1), jnp.float32)
    for i in range(T // 128):
        blk = oh[:, i * 128:(i + 1) * 128]
        c = jnp.dot(blk, upper, preferred_element_type=jnp.float32) + carry
        ranks_ref[:, i * 128:(i + 1) * 128] = c
        carry = c[:, 127:128]
    counts = carry

    il0 = lax.broadcasted_iota(jnp.int32, (E, E), 0)
    il1 = lax.broadcasted_iota(jnp.int32, (E, E), 1)
    strict = (il0 > il1).astype(jnp.float32)
    offs = jnp.dot(strict, counts, preferred_element_type=jnp.float32,
                   precision=lax.Precision.HIGHEST)

    dest_f = jnp.sum(oh * (offs + ranks_ref[...] - 1.0), axis=0, keepdims=True)
    dest_ref[...] = dest_f.astype(jnp.int32)
    offs9 = jnp.concatenate([jnp.zeros((1, 1), jnp.float32), offs + counts],
                            axis=0).astype(jnp.int32)
    offs_ref[...] = jnp.broadcast_to(offs9, (E + 1, 128))


_router = pl.pallas_call(
    _router_body,
    out_shape=[
        jax.ShapeDtypeStruct((1, T), jnp.int32),
        jax.ShapeDtypeStruct((1, T), jnp.float32),
        jax.ShapeDtypeStruct((E + 1, 128), jnp.int32),
    ],
    scratch_shapes=[pltpu.VMEM((E, T), jnp.float32)],
)


def _gmm_body(vs_ref, eid_ref, off_ref, x_ref, w_hbm, b_ref, o_ref,
              w_buf, sems):
    m = pl.program_id(0)
    v0 = vs_ref[m]
    v1 = vs_ref[m + 1]

    @pl.when(m == 0)
    def _():
        for ee in range(E):
            pltpu.make_async_copy(
                w_hbm.at[ee], w_buf.at[ee], sems.at[ee]).start()

    rows = m * TM + lax.broadcasted_iota(jnp.int32, (TM, 1), 0)

    def visit(v, carry):
        e = eid_ref[v]
        new_run = jnp.logical_or(v == 0, e != eid_ref[jnp.maximum(v - 1, 0)])

        @pl.when(new_run)
        def _():
            pltpu.make_async_copy(
                w_hbm.at[e], w_buf.at[e], sems.at[e]).wait()

        mask = (rows >= off_ref[e]) & (rows < off_ref[e + 1])
        y = jnp.dot(x_ref[...], w_buf[e], preferred_element_type=jnp.float32)
        y = jnp.maximum(y + b_ref[pl.ds(e, 1), :], 0.0)
        o_ref[...] = jnp.where(mask, y, o_ref[...])
        return carry

    lax.fori_loop(v0, v1, visit, 0)

    @pl.when(m == NTILES - 1)
    def _():
        for ee in range(E):
            @pl.when(off_ref[ee] == off_ref[ee + 1])
            def _():
                pltpu.make_async_copy(
                    w_hbm.at[ee], w_buf.at[ee], sems.at[ee]).wait()


_gmm = pl.pallas_call(
    _gmm_body,
    grid_spec=pltpu.PrefetchScalarGridSpec(
        num_scalar_prefetch=3,
        grid=(NTILES,),
        in_specs=[
            pl.BlockSpec((TM, H), lambda m, vs, eid, off: (m, 0)),
            pl.BlockSpec(memory_space=pl.ANY),
            pl.BlockSpec((E, H), lambda m, vs, eid, off: (0, 0)),
        ],
        out_specs=pl.BlockSpec((TM, H), lambda m, vs, eid, off: (m, 0)),
        scratch_shapes=[
            pltpu.VMEM((E, H, H), jnp.float32),
            pltpu.SemaphoreType.DMA((E,)),
        ],
    ),
    out_shape=jax.ShapeDtypeStruct((T, H), jnp.float32),
)


def _make_sc_kernels():
    mesh = plsc.VectorSubcoreMesh(core_axis_name="c", subcore_axis_name="s")

    HALFD = CHUNK // 2

    @functools.partial(
        pl.kernel,
        mesh=mesh,
        out_type=jax.ShapeDtypeStruct((T, H), jnp.float32),
        scratch_types=[
            pltpu.VMEM((HALFD,), jnp.int32),
            pltpu.VMEM((HALFD,), jnp.int32),
            pltpu.VMEM((HALFD, H), jnp.float32),
            pltpu.VMEM((HALFD, H), jnp.float32),
            pltpu.SemaphoreType.DMA,
            pltpu.SemaphoreType.DMA,
            pltpu.SemaphoreType.DMA,
            pltpu.SemaphoreType.DMA,
        ],
    )
    def dispatch(x_hbm, dest_hbm, xs_hbm,
                 idx0, idx1, rows0, rows1, l0s, l1s, s0s, s1s):
        wid = lax.axis_index("s") * 2 + lax.axis_index("c")
        base = wid * CHUNK
        l0 = pltpu.make_async_copy(x_hbm.at[pl.ds(base, HALFD)], rows0, l0s)
        l0.start()
        l1 = pltpu.make_async_copy(
            x_hbm.at[pl.ds(base + HALFD, HALFD)], rows1, l1s)
        l1.start()
        pltpu.sync_copy(dest_hbm.at[pl.ds(base, HALFD)], idx0)
        pltpu.sync_copy(dest_hbm.at[pl.ds(base + HALFD, HALFD)], idx1)
        l0.wait()
        s0 = pltpu.make_async_copy(rows0, xs_hbm.at[idx0], s0s)
        s0.start()
        l1.wait()
        s1 = pltpu.make_async_copy(rows1, xs_hbm.at[idx1], s1s)
        s1.start()
        s0.wait()
        s1.wait()

    HALF = CHUNK // 2

    @functools.partial(
        pl.kernel,
        mesh=mesh,
        out_type=jax.ShapeDtypeStruct((T, H), jnp.float32),
        scratch_types=[
            pltpu.VMEM((HALF,), jnp.int32),
            pltpu.VMEM((HALF,), jnp.int32),
            pltpu.VMEM((CHUNK,), jnp.float32),
            pltpu.VMEM((HALF, H), jnp.float32),
            pltpu.VMEM((HALF, H), jnp.float32),
            pltpu.SemaphoreType.DMA,
            pltpu.SemaphoreType.DMA,
            pltpu.SemaphoreType.DMA,
        ],
    )
    def combine(y_hbm, dest_hbm, prob_hbm, out_hbm,
                idx0, idx1, p_v, rows0, rows1, sem0, sem1, wsem):
        wid = lax.axis_index("s") * 2 + lax.axis_index("c")
        base = wid * CHUNK
        pltpu.sync_copy(dest_hbm.at[pl.ds(base, HALF)], idx0)
        g0 = pltpu.make_async_copy(y_hbm.at[idx0], rows0, sem0)
        g0.start()
        pltpu.sync_copy(dest_hbm.at[pl.ds(base + HALF, HALF)], idx1)
        g1 = pltpu.make_async_copy(y_hbm.at[idx1], rows1, sem1)
        g1.start()
        pltpu.sync_copy(prob_hbm.at[pl.ds(base, CHUNK)], p_v)

        def scale_group(rows_v, qoff):
            def body(q, acc):
                pv = p_v[pl.ds(qoff + q * LANES, LANES)]
                for j in range(LANES):
                    pr = jnp.broadcast_to(pv[j], (LANES,))
                    r = q * LANES + j
                    for c in range(H // LANES):
                        sl = pl.ds(c * LANES, LANES)
                        rows_v[r, sl] = rows_v[r, sl] * pr
                return acc
            return body

        g0.wait()
        lax.fori_loop(0, HALF // LANES, scale_group(rows0, 0), 0)
        w0 = pltpu.make_async_copy(rows0, out_hbm.at[pl.ds(base, HALF)], wsem)
        w0.start()
        g1.wait()
        lax.fori_loop(0, HALF // LANES, scale_group(rows1, HALF), 0)
        pltpu.sync_copy(rows1, out_hbm.at[pl.ds(base + HALF, HALF)])
        w0.wait()

    return dispatch, combine


_make_sc_kernels = functools.cache(_make_sc_kernels)


def kernel(input, gate, We, be):
    dest2, prob2, offs2 = _router(gate.T)
    dest = dest2.reshape(T)
    prob = prob2.reshape(T)
    offs = offs2[:, 0]

    first = offs[:E] // TM
    last = (offs[1:] - 1) // TM
    nv = jnp.maximum(last - first + 1, 0)
    cum = jnp.cumsum(nv)
    gidx = jnp.arange(G, dtype=jnp.int32)
    e_g = jnp.minimum(
        jnp.sum((cum[None, :] <= gidx[:, None]).astype(jnp.int32), axis=1),
        E - 1)
    t_g = jnp.clip(first[e_g] + gidx - (cum - nv)[e_g], 0, NTILES - 1)
    nvis = cum[E - 1]
    marr = jnp.arange(NTILES + 1, dtype=jnp.int32)
    vs = jnp.sum(((gidx[None, :] < nvis) & (t_g[None, :] < marr[:, None]))
                 .astype(jnp.int32), axis=1)

    dispatch, combine = _make_sc_kernels()
    xs = dispatch(input, dest)
    ys = _gmm(vs, e_g, offs, xs, We, be)
    return combine(ys, dest, prob)


# speedup vs baseline: 1.2184x; 1.0067x over previous
import functools

import jax
import jax.numpy as jnp
from jax import lax
from jax.experimental import pallas as pl
from jax.experimental.pallas import tpu as pltpu
from jax.experimental.pallas import tpu_sc as plsc

E = 8
H = 768
T = 2048
TM = 256
NTILES = T // TM
G = NTILES + E - 1
NW = 32
CHUNK = T // NW
LANES = 16


def _router_body(gate_t_ref, dest_ref, prob_ref, offs_ref, ranks_ref):
    gate_t = gate_t_ref[...]
    mx = jnp.max(gate_t, axis=0, keepdims=True)
    s = jnp.sum(jnp.exp(gate_t - mx), axis=0, keepdims=True)
    prob_ref[...] = 1.0 / s

    ioe = lax.broadcasted_iota(jnp.int32, (E, T), 0)
    idx = jnp.min(jnp.where(gate_t == mx, ioe, E), axis=0, keepdims=True)
    oh = (ioe == idx).astype(jnp.float32)

    iu0 = lax.broadcasted_iota(jnp.int32, (128, 128), 0)
    iu1 = lax.broadcasted_iota(jnp.int32, (128, 128), 1)
    upper = (iu0 <= iu1).astype(jnp.float32)
    NB = T // 128
    local = [jnp.dot(oh[:, i * 128:(i + 1) * 128], upper,
                     preferred_element_type=jnp.float32) for i in range(NB)]
    totals = jnp.concatenate([l[:, 127:128] for l in local], axis=1)
    ib0 = lax.broadcasted_iota(jnp.int32, (NB, NB), 0)
    ib1 = lax.broadcasted_iota(jnp.int32, (NB, NB), 1)
    prefix = jnp.dot(totals, (ib0 < ib1).astype(jnp.float32),
                     preferred_element_type=jnp.float32,
                     precision=lax.Precision.HIGHEST)
    for i in range(NB):
        ranks_ref[:, i * 128:(i + 1) * 128] = local[i] + prefix[:, i:i + 1]
    counts = prefix[:, NB - 1:NB] + totals[:, NB - 1:NB]

    il0 = lax.broadcasted_iota(jnp.int32, (E, E), 0)
    il1 = lax.broadcasted_iota(jnp.int32, (E, E), 1)
    strict = (il0 > il1).astype(jnp.float32)
    offs = jnp.dot(strict, counts, preferred_element_type=jnp.float32,
                   precision=lax.Precision.HIGHEST)

    dest_f = jnp.sum(oh * (offs + ranks_ref[...] - 1.0), axis=0, keepdims=True)
    dest_ref[...] = dest_f.astype(jnp.int32)
    offs9 = jnp.concatenate([jnp.zeros((1, 1), jnp.float32), offs + counts],
                            axis=0).astype(jnp.int32)
    offs_ref[...] = jnp.broadcast_to(offs9, (E + 1, 128))


_router = pl.pallas_call(
    _router_body,
    out_shape=[
        jax.ShapeDtypeStruct((1, T), jnp.int32),
        jax.ShapeDtypeStruct((1, T), jnp.float32),
        jax.ShapeDtypeStruct((E + 1, 128), jnp.int32),
    ],
    scratch_shapes=[pltpu.VMEM((E, T), jnp.float32)],
)


def _gmm_body(vs_ref, eid_ref, off_ref, x_ref, w_hbm, b_ref, o_ref,
              w_buf, sems):
    m = pl.program_id(0)
    v0 = vs_ref[m]
    v1 = vs_ref[m + 1]

    @pl.when(m == 0)
    def _():
        for ee in range(E):
            pltpu.make_async_copy(
                w_hbm.at[ee], w_buf.at[ee], sems.at[ee]).start()

    rows = m * TM + lax.broadcasted_iota(jnp.int32, (TM, 1), 0)

    def visit(v, carry):
        e = eid_ref[v]
        new_run = jnp.logical_or(v == 0, e != eid_ref[jnp.maximum(v - 1, 0)])

        @pl.when(new_run)
        def _():
            pltpu.make_async_copy(
                w_hbm.at[e], w_buf.at[e], sems.at[e]).wait()

        mask = (rows >= off_ref[e]) & (rows < off_ref[e + 1])
        y = jnp.dot(x_ref[...], w_buf[e], preferred_element_type=jnp.float32)
        y = jnp.maximum(y + b_ref[pl.ds(e, 1), :], 0.0)
        o_ref[...] = jnp.where(mask, y, o_ref[...])
        return carry

    lax.fori_loop(v0, v1, visit, 0)

    @pl.when(m == NTILES - 1)
    def _():
        for ee in range(E):
            @pl.when(off_ref[ee] == off_ref[ee + 1])
            def _():
                pltpu.make_async_copy(
                    w_hbm.at[ee], w_buf.at[ee], sems.at[ee]).wait()


_gmm = pl.pallas_call(
    _gmm_body,
    grid_spec=pltpu.PrefetchScalarGridSpec(
        num_scalar_prefetch=3,
        grid=(NTILES,),
        in_specs=[
            pl.BlockSpec((TM, H), lambda m, vs, eid, off: (m, 0)),
            pl.BlockSpec(memory_space=pl.ANY),
            pl.BlockSpec((E, H), lambda m, vs, eid, off: (0, 0)),
        ],
        out_specs=pl.BlockSpec((TM, H), lambda m, vs, eid, off: (m, 0)),
        scratch_shapes=[
            pltpu.VMEM((E, H, H), jnp.float32),
            pltpu.SemaphoreType.DMA((E,)),
        ],
    ),
    out_shape=jax.ShapeDtypeStruct((T, H), jnp.float32),
)


def _make_sc_kernels():
    mesh = plsc.VectorSubcoreMesh(core_axis_name="c", subcore_axis_name="s")

    HALFD = CHUNK // 2

    @functools.partial(
        pl.kernel,
        mesh=mesh,
        out_type=jax.ShapeDtypeStruct((T, H), jnp.float32),
        scratch_types=[
            pltpu.VMEM((HALFD,), jnp.int32),
            pltpu.VMEM((HALFD,), jnp.int32),
            pltpu.VMEM((HALFD, H), jnp.float32),
            pltpu.VMEM((HALFD, H), jnp.float32),
            pltpu.SemaphoreType.DMA,
            pltpu.SemaphoreType.DMA,
            pltpu.SemaphoreType.DMA,
            pltpu.SemaphoreType.DMA,
        ],
    )
    def dispatch(x_hbm, dest_hbm, xs_hbm,
                 idx0, idx1, rows0, rows1, l0s, l1s, s0s, s1s):
        wid = lax.axis_index("s") * 2 + lax.axis_index("c")
        base = wid * CHUNK
        l0 = pltpu.make_async_copy(x_hbm.at[pl.ds(base, HALFD)], rows0, l0s)
        l0.start()
        l1 = pltpu.make_async_copy(
            x_hbm.at[pl.ds(base + HALFD, HALFD)], rows1, l1s)
        l1.start()
        pltpu.sync_copy(dest_hbm.at[pl.ds(base, HALFD)], idx0)
        pltpu.sync_copy(dest_hbm.at[pl.ds(base + HALFD, HALFD)], idx1)
        l0.wait()
        s0 = pltpu.make_async_copy(rows0, xs_hbm.at[idx0], s0s)
        s0.start()
        l1.wait()
        s1 = pltpu.make_async_copy(rows1, xs_hbm.at[idx1], s1s)
        s1.start()
        s0.wait()
        s1.wait()

    HALF = CHUNK // 2

    @functools.partial(
        pl.kernel,
        mesh=mesh,
        out_type=jax.ShapeDtypeStruct((T, H), jnp.float32),
        scratch_types=[
            pltpu.VMEM((HALF,), jnp.int32),
            pltpu.VMEM((HALF,), jnp.int32),
            pltpu.VMEM((CHUNK,), jnp.float32),
            pltpu.VMEM((HALF, H), jnp.float32),
            pltpu.VMEM((HALF, H), jnp.float32),
            pltpu.SemaphoreType.DMA,
            pltpu.SemaphoreType.DMA,
            pltpu.SemaphoreType.DMA,
        ],
    )
    def combine(y_hbm, dest_hbm, prob_hbm, out_hbm,
                idx0, idx1, p_v, rows0, rows1, sem0, sem1, wsem):
        wid = lax.axis_index("s") * 2 + lax.axis_index("c")
        base = wid * CHUNK
        pltpu.sync_copy(dest_hbm.at[pl.ds(base, HALF)], idx0)
        g0 = pltpu.make_async_copy(y_hbm.at[idx0], rows0, sem0)
        g0.start()
        pltpu.sync_copy(dest_hbm.at[pl.ds(base + HALF, HALF)], idx1)
        g1 = pltpu.make_async_copy(y_hbm.at[idx1], rows1, sem1)
        g1.start()
        pltpu.sync_copy(prob_hbm.at[pl.ds(base, CHUNK)], p_v)

        def scale_group(rows_v, qoff):
            def body(q, acc):
                pv = p_v[pl.ds(qoff + q * LANES, LANES)]
                for j in range(LANES):
                    pr = jnp.broadcast_to(pv[j], (LANES,))
                    r = q * LANES + j
                    for c in range(H // LANES):
                        sl = pl.ds(c * LANES, LANES)
                        rows_v[r, sl] = rows_v[r, sl] * pr
                return acc
            return body

        g0.wait()
        lax.fori_loop(0, HALF // LANES, scale_group(rows0, 0), 0)
        w0 = pltpu.make_async_copy(rows0, out_hbm.at[pl.ds(base, HALF)], wsem)
        w0.start()
        g1.wait()
        lax.fori_loop(0, HALF // LANES, scale_group(rows1, HALF), 0)
        pltpu.sync_copy(rows1, out_hbm.at[pl.ds(base + HALF, HALF)])
        w0.wait()

    return dispatch, combine


_make_sc_kernels = functools.cache(_make_sc_kernels)


def kernel(input, gate, We, be):
    dest2, prob2, offs2 = _router(gate.T)
    dest = dest2.reshape(T)
    prob = prob2.reshape(T)
    offs = offs2[:, 0]

    first = offs[:E] // TM
    last = (offs[1:] - 1) // TM
    nv = jnp.maximum(last - first + 1, 0)
    cum = jnp.cumsum(nv)
    gidx = jnp.arange(G, dtype=jnp.int32)
    e_g = jnp.minimum(
        jnp.sum((cum[None, :] <= gidx[:, None]).astype(jnp.int32), axis=1),
        E - 1)
    t_g = jnp.clip(first[e_g] + gidx - (cum - nv)[e_g], 0, NTILES - 1)
    nvis = cum[E - 1]
    marr = jnp.arange(NTILES + 1, dtype=jnp.int32)
    vs = jnp.sum(((gidx[None, :] < nvis) & (t_g[None, :] < marr[:, None]))
                 .astype(jnp.int32), axis=1)

    dispatch, combine = _make_sc_kernels()
    xs = dispatch(input, dest)
    ys = _gmm(vs, e_g, offs, xs, We, be)
    return combine(ys, dest, prob)
